# Initial kernel scaffold; baseline (speedup 1.0000x reference)
import jax
import jax.numpy as jnp
from jax.experimental import pallas as pl


def kernel(x, edge_index, W1, b1, W2, b2):
    raise NotImplementedError("write your pallas kernel here")


# speedup vs baseline: 21.3360x; 21.3360x over previous
import functools

import jax
import jax.numpy as jnp
from jax import lax
from jax.experimental import pallas as pl
from jax.experimental.pallas import tpu as pltpu
from jax.experimental.pallas import tpu_sc as plsc

NC = 2
NS = 16
NW = NC * NS
K = 128


def _mesh():
    return plsc.VectorSubcoreMesh(core_axis_name="c", subcore_axis_name="s",
                                  num_cores=NC, num_subcores=NS)


def _make_deg_kernel(R, NCH, rps):
    @functools.partial(
        pl.kernel,
        out_type=jax.ShapeDtypeStruct((NC, R, 16), jnp.float32),
        mesh=_mesh(),
        scratch_types=[
            pltpu.VMEM((NCH, K), jnp.int32),
            pltpu.VMEM((K, 16), jnp.float32),
            pltpu.VMEM_SHARED((R, 16), jnp.float32),
        ],
        compiler_params=pltpu.CompilerParams(use_tc_tiling_on_sc=False),
    )
    def deg_k(dst_hbm, zeros_hbm, ones_hbm, out_hbm, dst_v, ones_v, acc_sh):
        c = lax.axis_index("c")
        s = lax.axis_index("s")
        wid = c * NS + s
        pltpu.sync_copy(zeros_hbm, acc_sh.at[pl.ds(s * rps, rps)])
        pltpu.sync_copy(ones_hbm, ones_v)
        pltpu.sync_copy(dst_hbm.at[wid], dst_v)
        plsc.subcore_barrier()

        def step(j, carry):
            pltpu.sync_copy(ones_v, acc_sh.at[dst_v.at[j]], add=True)
            return carry

        lax.fori_loop(0, NCH, step, 0)
        plsc.subcore_barrier()
        pltpu.sync_copy(
            acc_sh.at[pl.ds(s * rps, rps)],
            out_hbm.at[c, pl.ds(s * rps, rps)],
        )

    return deg_k


def _make_agg_kernel(D, R, NCH, rps):
    @functools.partial(
        pl.kernel,
        out_type=jax.ShapeDtypeStruct((NC, R, D), jnp.float32),
        mesh=_mesh(),
        scratch_types=[
            pltpu.VMEM((NCH, K), jnp.int32),
            pltpu.VMEM((NCH, K), jnp.int32),
            pltpu.VMEM((2, K, D), jnp.float32),
            pltpu.VMEM_SHARED((R, D), jnp.float32),
            pltpu.SemaphoreType.DMA,
            pltpu.SemaphoreType.DMA,
        ],
        compiler_params=pltpu.CompilerParams(use_tc_tiling_on_sc=False),
    )
    def agg_k(h_hbm, src_hbm, dst_hbm, zeros_hbm, out_hbm,
              src_v, dst_v, rows_v, acc_sh, sem0, sem1):
        c = lax.axis_index("c")
        s = lax.axis_index("s")
        wid = c * NS + s
        pltpu.sync_copy(zeros_hbm, acc_sh.at[pl.ds(s * rps, rps)])
        pltpu.sync_copy(src_hbm.at[wid], src_v)
        pltpu.sync_copy(dst_hbm.at[wid], dst_v)
        plsc.subcore_barrier()

        def step(j, carry):
            pltpu.async_copy(h_hbm.at[src_v.at[j]], rows_v.at[0], sem0).wait()
            pltpu.sync_copy(rows_v.at[0], acc_sh.at[dst_v.at[j]], add=True)
            return carry

        lax.fori_loop(0, NCH, step, 0)
        plsc.subcore_barrier()
        pltpu.sync_copy(
            acc_sh.at[pl.ds(s * rps, rps)],
            out_hbm.at[c, pl.ds(s * rps, rps)],
        )

    return agg_k


def _tc_matmul(x, W, bn):
    n, d_in = x.shape
    d_out = W.shape[1]

    def mm(x_ref, w_ref, o_ref):
        o_ref[...] = jnp.dot(x_ref[...], w_ref[...],
                             preferred_element_type=jnp.float32)

    return pl.pallas_call(
        mm,
        grid=(n // bn,),
        in_specs=[
            pl.BlockSpec((bn, d_in), lambda i: (i, 0)),
            pl.BlockSpec((d_in, d_out), lambda i: (0, 0)),
        ],
        out_specs=pl.BlockSpec((bn, d_out), lambda i: (i, 0)),
        out_shape=jax.ShapeDtypeStruct((n, d_out), jnp.float32),
    )(x, W)


def _tc_scale(d0, d1, h1, bn):
    n, d = h1.shape

    def body(d0_ref, d1_ref, h1_ref, dis_ref, h1s_ref):
        deg = d0_ref[...] + d1_ref[...] + 1.0
        dis = lax.rsqrt(deg)
        dis_ref[...] = dis
        h1s_ref[...] = dis * h1_ref[...]

    return pl.pallas_call(
        body,
        grid=(n // bn,),
        in_specs=[
            pl.BlockSpec((bn, 1), lambda i: (i, 0)),
            pl.BlockSpec((bn, 1), lambda i: (i, 0)),
            pl.BlockSpec((bn, d), lambda i: (i, 0)),
        ],
        out_specs=[
            pl.BlockSpec((bn, 1), lambda i: (i, 0)),
            pl.BlockSpec((bn, d), lambda i: (i, 0)),
        ],
        out_shape=[
            jax.ShapeDtypeStruct((n, 1), jnp.float32),
            jax.ShapeDtypeStruct((n, d), jnp.float32),
        ],
    )(d0, d1, h1)


def _tc_layer2(p0, p1, h1s, dis, b1, W2, bn):
    n, d = h1s.shape
    d_out = W2.shape[1]

    def body(p0_ref, p1_ref, h1s_ref, dis_ref, b1_ref, w2_ref, o_ref):
        dis = dis_ref[...]
        z = dis * (p0_ref[...] + p1_ref[...] + h1s_ref[...]) + b1_ref[...]
        z = jnp.maximum(z, 0.0)
        o_ref[...] = dis * jnp.dot(z, w2_ref[...],
                                   preferred_element_type=jnp.float32)

    return pl.pallas_call(
        body,
        grid=(n // bn,),
        in_specs=[
            pl.BlockSpec((bn, d), lambda i: (i, 0)),
            pl.BlockSpec((bn, d), lambda i: (i, 0)),
            pl.BlockSpec((bn, d), lambda i: (i, 0)),
            pl.BlockSpec((bn, 1), lambda i: (i, 0)),
            pl.BlockSpec((1, d), lambda i: (0, 0)),
            pl.BlockSpec((d, d_out), lambda i: (0, 0)),
        ],
        out_specs=pl.BlockSpec((bn, d_out), lambda i: (i, 0)),
        out_shape=jax.ShapeDtypeStruct((n, d_out), jnp.float32),
    )(p0, p1, h1s, dis, b1, W2)


def _tc_final(p0, p1, h2s, dis, b2, bn):
    n, d = h2s.shape

    def body(p0_ref, p1_ref, h2s_ref, dis_ref, b2_ref, o_ref):
        o_ref[...] = dis_ref[...] * (
            p0_ref[...] + p1_ref[...] + h2s_ref[...]) + b2_ref[...]

    return pl.pallas_call(
        body,
        grid=(n // bn,),
        in_specs=[
            pl.BlockSpec((bn, d), lambda i: (i, 0)),
            pl.BlockSpec((bn, d), lambda i: (i, 0)),
            pl.BlockSpec((bn, d), lambda i: (i, 0)),
            pl.BlockSpec((bn, 1), lambda i: (i, 0)),
            pl.BlockSpec((1, d), lambda i: (0, 0)),
        ],
        out_specs=pl.BlockSpec((bn, d), lambda i: (i, 0)),
        out_shape=jax.ShapeDtypeStruct((n, d), jnp.float32),
    )(p0, p1, h2s, dis, b2)


def kernel(x, edge_index, W1, b1, W2, b2):
    n = x.shape[0]
    e = edge_index.shape[1]
    bn = 1000 if n % 1000 == 0 else 8

    nch = -(-e // (NW * K))
    ep = NW * nch * K
    rps = -(-(n + 1) // NS)
    rps = -(-rps // 8) * 8
    R = NS * rps

    src = edge_index[0].astype(jnp.int32)
    dst = edge_index[1].astype(jnp.int32)
    pad = ep - e
    src3 = jnp.concatenate([src, jnp.zeros((pad,), jnp.int32)]).reshape(NW, nch, K)
    dst3 = jnp.concatenate([dst, jnp.full((pad,), n, jnp.int32)]).reshape(NW, nch, K)

    zeros16 = jnp.zeros((rps, 16), jnp.float32)
    ones16 = jnp.ones((K, 16), jnp.float32)

    deg_parts = _make_deg_kernel(R, nch, rps)(dst3, zeros16, ones16)
    h1 = _tc_matmul(x, W1, bn)

    d0 = deg_parts[0, :n, 0:1]
    d1 = deg_parts[1, :n, 0:1]
    dis, h1s = _tc_scale(d0, d1, h1, bn)

    d_hid = W1.shape[1]
    zeros_h = jnp.zeros((rps, d_hid), jnp.float32)
    parts1 = _make_agg_kernel(d_hid, R, nch, rps)(h1s, src3, dst3, zeros_h)
    h2s = _tc_layer2(parts1[0, :n], parts1[1, :n], h1s, dis,
                     b1.reshape(1, -1), W2, bn)

    d_out = W2.shape[1]
    zeros_o = jnp.zeros((rps, d_out), jnp.float32)
    parts2 = _make_agg_kernel(d_out, R, nch, rps)(h2s, src3, dst3, zeros_o)
    out = _tc_final(parts2[0, :n], parts2[1, :n], h2s, dis,
                    b2.reshape(1, -1), bn)
    return out


# speedup vs baseline: 23.4173x; 1.0975x over previous
import functools

import jax
import jax.numpy as jnp
from jax import lax
from jax.experimental import pallas as pl
from jax.experimental.pallas import tpu as pltpu
from jax.experimental.pallas import tpu_sc as plsc

NC = 2
NS = 16
NW = NC * NS
K = 128


def _mesh():
    return plsc.VectorSubcoreMesh(core_axis_name="c", subcore_axis_name="s",
                                  num_cores=NC, num_subcores=NS)


def _make_deg_kernel(R, NCH, rps):
    @functools.partial(
        pl.kernel,
        out_type=jax.ShapeDtypeStruct((NC, R, 16), jnp.float32),
        mesh=_mesh(),
        scratch_types=[
            pltpu.VMEM((NCH, K), jnp.int32),
            pltpu.VMEM((K, 16), jnp.float32),
            pltpu.VMEM_SHARED((R, 16), jnp.float32),
        ],
        compiler_params=pltpu.CompilerParams(use_tc_tiling_on_sc=False),
    )
    def deg_k(dst_hbm, zeros_hbm, ones_hbm, out_hbm, dst_v, ones_v, acc_sh):
        c = lax.axis_index("c")
        s = lax.axis_index("s")
        wid = c * NS + s
        pltpu.sync_copy(zeros_hbm, acc_sh.at[pl.ds(s * rps, rps)])
        pltpu.sync_copy(ones_hbm, ones_v)
        pltpu.sync_copy(dst_hbm.at[wid], dst_v)
        plsc.subcore_barrier()

        def step(j, carry):
            pltpu.sync_copy(ones_v, acc_sh.at[dst_v.at[j]], add=True)
            return carry

        lax.fori_loop(0, NCH, step, 0)
        plsc.subcore_barrier()
        pltpu.sync_copy(
            acc_sh.at[pl.ds(s * rps, rps)],
            out_hbm.at[c, pl.ds(s * rps, rps)],
        )

    return deg_k


def _make_agg_kernel(D, R, NCH, rps):
    @functools.partial(
        pl.kernel,
        out_type=jax.ShapeDtypeStruct((NC, R, D), jnp.float32),
        mesh=_mesh(),
        scratch_types=[
            pltpu.VMEM((NCH, K), jnp.int32),
            pltpu.VMEM((NCH, K), jnp.int32),
            pltpu.VMEM((2, K, D), jnp.float32),
            pltpu.VMEM_SHARED((R, D), jnp.float32),
            pltpu.SemaphoreType.DMA,
            pltpu.SemaphoreType.DMA,
        ],
        compiler_params=pltpu.CompilerParams(use_tc_tiling_on_sc=False),
    )
    def agg_k(h_hbm, src_hbm, dst_hbm, zeros_hbm, out_hbm,
              src_v, dst_v, rows_v, acc_sh, sem0, sem1):
        c = lax.axis_index("c")
        s = lax.axis_index("s")
        wid = c * NS + s
        pltpu.sync_copy(zeros_hbm, acc_sh.at[pl.ds(s * rps, rps)])
        pltpu.sync_copy(src_hbm.at[wid], src_v)
        pltpu.sync_copy(dst_hbm.at[wid], dst_v)
        plsc.subcore_barrier()

        pltpu.async_copy(h_hbm.at[src_v.at[0]], rows_v.at[0], sem0)

        def step(j, carry):
            buf = lax.rem(j, 2)
            nbuf = 1 - buf
            pltpu.make_async_copy(h_hbm.at[src_v.at[j]],
                                  rows_v.at[buf], sem0).wait()

            @pl.when(j >= 1)
            def _():
                pltpu.make_async_copy(rows_v.at[nbuf],
                                      acc_sh.at[dst_v.at[j - 1]], sem1).wait()

            @pl.when(j < NCH - 1)
            def _():
                pltpu.async_copy(h_hbm.at[src_v.at[j + 1]],
                                 rows_v.at[nbuf], sem0)

            pltpu.async_copy(rows_v.at[buf], acc_sh.at[dst_v.at[j]], sem1,
                             add=True)
            return carry

        lax.fori_loop(0, NCH, step, 0)
        last = NCH - 1
        pltpu.make_async_copy(rows_v.at[lax.rem(last, 2)],
                              acc_sh.at[dst_v.at[last]], sem1).wait()
        plsc.subcore_barrier()
        pltpu.sync_copy(
            acc_sh.at[pl.ds(s * rps, rps)],
            out_hbm.at[c, pl.ds(s * rps, rps)],
        )

    return agg_k


def _tc_matmul(x, W, bn):
    n, d_in = x.shape
    d_out = W.shape[1]

    def mm(x_ref, w_ref, o_ref):
        o_ref[...] = jnp.dot(x_ref[...], w_ref[...],
                             preferred_element_type=jnp.float32)

    return pl.pallas_call(
        mm,
        grid=(n // bn,),
        in_specs=[
            pl.BlockSpec((bn, d_in), lambda i: (i, 0)),
            pl.BlockSpec((d_in, d_out), lambda i: (0, 0)),
        ],
        out_specs=pl.BlockSpec((bn, d_out), lambda i: (i, 0)),
        out_shape=jax.ShapeDtypeStruct((n, d_out), jnp.float32),
    )(x, W)


def _tc_scale(d0, d1, h1, bn):
    n, d = h1.shape

    def body(d0_ref, d1_ref, h1_ref, dis_ref, h1s_ref):
        deg = d0_ref[...] + d1_ref[...] + 1.0
        dis = lax.rsqrt(deg)
        dis_ref[...] = dis
        h1s_ref[...] = dis * h1_ref[...]

    return pl.pallas_call(
        body,
        grid=(n // bn,),
        in_specs=[
            pl.BlockSpec((bn, 1), lambda i: (i, 0)),
            pl.BlockSpec((bn, 1), lambda i: (i, 0)),
            pl.BlockSpec((bn, d), lambda i: (i, 0)),
        ],
        out_specs=[
            pl.BlockSpec((bn, 1), lambda i: (i, 0)),
            pl.BlockSpec((bn, d), lambda i: (i, 0)),
        ],
        out_shape=[
            jax.ShapeDtypeStruct((n, 1), jnp.float32),
            jax.ShapeDtypeStruct((n, d), jnp.float32),
        ],
    )(d0, d1, h1)


def _tc_layer2(p0, p1, h1s, dis, b1, W2, bn):
    n, d = h1s.shape
    d_out = W2.shape[1]

    def body(p0_ref, p1_ref, h1s_ref, dis_ref, b1_ref, w2_ref, o_ref):
        dis = dis_ref[...]
        z = dis * (p0_ref[...] + p1_ref[...] + h1s_ref[...]) + b1_ref[...]
        z = jnp.maximum(z, 0.0)
        o_ref[...] = dis * jnp.dot(z, w2_ref[...],
                                   preferred_element_type=jnp.float32)

    return pl.pallas_call(
        body,
        grid=(n // bn,),
        in_specs=[
            pl.BlockSpec((bn, d), lambda i: (i, 0)),
            pl.BlockSpec((bn, d), lambda i: (i, 0)),
            pl.BlockSpec((bn, d), lambda i: (i, 0)),
            pl.BlockSpec((bn, 1), lambda i: (i, 0)),
            pl.BlockSpec((1, d), lambda i: (0, 0)),
            pl.BlockSpec((d, d_out), lambda i: (0, 0)),
        ],
        out_specs=pl.BlockSpec((bn, d_out), lambda i: (i, 0)),
        out_shape=jax.ShapeDtypeStruct((n, d_out), jnp.float32),
    )(p0, p1, h1s, dis, b1, W2)


def _tc_final(p0, p1, h2s, dis, b2, bn):
    n, d = h2s.shape

    def body(p0_ref, p1_ref, h2s_ref, dis_ref, b2_ref, o_ref):
        o_ref[...] = dis_ref[...] * (
            p0_ref[...] + p1_ref[...] + h2s_ref[...]) + b2_ref[...]

    return pl.pallas_call(
        body,
        grid=(n // bn,),
        in_specs=[
            pl.BlockSpec((bn, d), lambda i: (i, 0)),
            pl.BlockSpec((bn, d), lambda i: (i, 0)),
            pl.BlockSpec((bn, d), lambda i: (i, 0)),
            pl.BlockSpec((bn, 1), lambda i: (i, 0)),
            pl.BlockSpec((1, d), lambda i: (0, 0)),
        ],
        out_specs=pl.BlockSpec((bn, d), lambda i: (i, 0)),
        out_shape=jax.ShapeDtypeStruct((n, d), jnp.float32),
    )(p0, p1, h2s, dis, b2)


def kernel(x, edge_index, W1, b1, W2, b2):
    n = x.shape[0]
    e = edge_index.shape[1]
    bn = 1000 if n % 1000 == 0 else 8

    nch = -(-e // (NW * K))
    ep = NW * nch * K
    rps = -(-(n + 1) // NS)
    rps = -(-rps // 8) * 8
    R = NS * rps

    src = edge_index[0].astype(jnp.int32)
    dst = edge_index[1].astype(jnp.int32)
    pad = ep - e
    src3 = jnp.concatenate([src, jnp.zeros((pad,), jnp.int32)]).reshape(NW, nch, K)
    dst3 = jnp.concatenate([dst, jnp.full((pad,), n, jnp.int32)]).reshape(NW, nch, K)

    zeros16 = jnp.zeros((rps, 16), jnp.float32)
    ones16 = jnp.ones((K, 16), jnp.float32)

    deg_parts = _make_deg_kernel(R, nch, rps)(dst3, zeros16, ones16)
    h1 = _tc_matmul(x, W1, bn)

    d0 = deg_parts[0, :n, 0:1]
    d1 = deg_parts[1, :n, 0:1]
    dis, h1s = _tc_scale(d0, d1, h1, bn)

    d_hid = W1.shape[1]
    zeros_h = jnp.zeros((rps, d_hid), jnp.float32)
    parts1 = _make_agg_kernel(d_hid, R, nch, rps)(h1s, src3, dst3, zeros_h)
    h2s = _tc_layer2(parts1[0, :n], parts1[1, :n], h1s, dis,
                     b1.reshape(1, -1), W2, bn)

    d_out = W2.shape[1]
    zeros_o = jnp.zeros((rps, d_out), jnp.float32)
    parts2 = _make_agg_kernel(d_out, R, nch, rps)(h2s, src3, dst3, zeros_o)
    out = _tc_final(parts2[0, :n], parts2[1, :n], h2s, dis,
                    b2.reshape(1, -1), bn)
    return out
